# Initial kernel scaffold; baseline (speedup 1.0000x reference)
import jax
import jax.numpy as jnp
from jax.experimental import pallas as pl


def kernel(x, pos_emb):
    raise NotImplementedError("write your pallas kernel here")


# speedup vs baseline: 2.4348x; 2.4348x over previous
import jax
import jax.numpy as jnp
from jax.experimental import pallas as pl


_BS = 512


def _add_kernel(x_ref, p_ref, o_ref):
    o_ref[...] = x_ref[...] + p_ref[...]


def kernel(x, pos_emb):
    b, s, d = x.shape
    grid = (b, s // _BS)
    return pl.pallas_call(
        _add_kernel,
        grid=grid,
        in_specs=[
            pl.BlockSpec((1, _BS, d), lambda i, j: (i, j, 0)),
            pl.BlockSpec((_BS, d), lambda i, j: (j, 0)),
        ],
        out_specs=pl.BlockSpec((1, _BS, d), lambda i, j: (i, j, 0)),
        out_shape=jax.ShapeDtypeStruct((b, s, d), x.dtype),
    )(x, pos_emb[:s])


# speedup vs baseline: 2.8635x; 1.1761x over previous
import jax
import jax.numpy as jnp
from jax.experimental import pallas as pl


_BS = 512


def _add_kernel(x_ref, p_ref, o_ref):
    o_ref[...] = x_ref[...] + p_ref[...]


def kernel(x, pos_emb):
    b, s, d = x.shape
    grid = (s // _BS, b)
    return pl.pallas_call(
        _add_kernel,
        grid=grid,
        in_specs=[
            pl.BlockSpec((1, _BS, d), lambda j, i: (i, j, 0)),
            pl.BlockSpec((_BS, d), lambda j, i: (j, 0)),
        ],
        out_specs=pl.BlockSpec((1, _BS, d), lambda j, i: (i, j, 0)),
        out_shape=jax.ShapeDtypeStruct((b, s, d), x.dtype),
    )(x, pos_emb[:s])


# speedup vs baseline: 3.1756x; 1.1090x over previous
import jax
import jax.numpy as jnp
from jax.experimental import pallas as pl


_BS = 1024


def _add_kernel(x_ref, p_ref, o_ref):
    o_ref[...] = x_ref[...] + p_ref[...]


def kernel(x, pos_emb):
    b, s, d = x.shape
    grid = (s // _BS, b)
    return pl.pallas_call(
        _add_kernel,
        grid=grid,
        in_specs=[
            pl.BlockSpec((1, _BS, d), lambda j, i: (i, j, 0)),
            pl.BlockSpec((_BS, d), lambda j, i: (j, 0)),
        ],
        out_specs=pl.BlockSpec((1, _BS, d), lambda j, i: (i, j, 0)),
        out_shape=jax.ShapeDtypeStruct((b, s, d), x.dtype),
    )(x, pos_emb[:s])


# speedup vs baseline: 3.3116x; 1.0428x over previous
import jax
import jax.numpy as jnp
from jax.experimental import pallas as pl


_BS = 2048


def _add_kernel(x_ref, p_ref, o_ref):
    o_ref[...] = x_ref[...] + p_ref[...]


def kernel(x, pos_emb):
    b, s, d = x.shape
    grid = (s // _BS, b)
    return pl.pallas_call(
        _add_kernel,
        grid=grid,
        in_specs=[
            pl.BlockSpec((1, _BS, d), lambda j, i: (i, j, 0)),
            pl.BlockSpec((_BS, d), lambda j, i: (j, 0)),
        ],
        out_specs=pl.BlockSpec((1, _BS, d), lambda j, i: (i, j, 0)),
        out_shape=jax.ShapeDtypeStruct((b, s, d), x.dtype),
    )(x, pos_emb[:s])
